# Initial kernel scaffold; baseline (speedup 1.0000x reference)
import jax
import jax.numpy as jnp
from jax.experimental import pallas as pl


def kernel(x, tables):
    raise NotImplementedError("write your pallas kernel here")


# speedup vs baseline: 1.4761x; 1.4761x over previous
import functools

import jax
import jax.numpy as jnp
from jax import lax
from jax.experimental import pallas as pl
from jax.experimental.pallas import tpu as pltpu
from jax.experimental.pallas import tpu_sc as plsc

N_CATS = 26
N_PLAIN = 22
N_TABLES = 23
VOCAB = 100000
DIM = 32
B, T = 1024, 50
GROUP_COLS = (22, 23, 24, 25)

NC, NS = 2, 16
NW = NC * NS
POS = B * T
CHUNK = POS // NW
LANES = 16


def _body(xt_hbm, tbl_hbm, *refs):
    outs = refs[:N_TABLES]
    idx_v, rows_a, rows_b, sem = refs[N_TABLES:]
    wid = lax.axis_index("s") * NC + lax.axis_index("c")
    base = wid * CHUNK

    def lookup(f, rows):
        pltpu.sync_copy(xt_hbm.at[pl.ds(f * POS + base, CHUNK)], idx_v)
        pltpu.async_copy(tbl_hbm.at[idx_v], rows, sem).wait()

    for f in range(N_PLAIN):
        lookup(f, rows_a)
        pltpu.sync_copy(rows_a, outs[f].at[pl.ds(base, CHUNK)])

    lookup(N_PLAIN, rows_a)
    for j in range(1, 4):
        lookup(N_PLAIN + j, rows_b)

        def add_row(i, _):
            for k in range(DIM // LANES):
                s = pl.ds(k * LANES, LANES)
                rows_a[i, s] = rows_a[i, s] + rows_b[i, s]
            return 0

        lax.fori_loop(0, CHUNK, add_row, 0)
    pltpu.sync_copy(rows_a, outs[N_PLAIN].at[pl.ds(base, CHUNK)])


@jax.jit
def kernel(x, tables):
    xt = jnp.transpose(x.reshape(POS, N_CATS))
    tid = jnp.array(
        list(range(N_PLAIN)) + [N_PLAIN] * 4, dtype=jnp.int32
    )
    xt = (xt + tid[:, None] * VOCAB).reshape(-1)
    tbl = tables.reshape(N_TABLES * VOCAB, DIM)

    mesh = plsc.VectorSubcoreMesh(core_axis_name="c", subcore_axis_name="s")
    out_type = tuple(
        jax.ShapeDtypeStruct((POS, DIM), jnp.float32) for _ in range(N_TABLES)
    )
    outs = pl.kernel(
        _body,
        out_type=out_type,
        mesh=mesh,
        scratch_types=[
            pltpu.VMEM((CHUNK,), jnp.int32),
            pltpu.VMEM((CHUNK, DIM), jnp.float32),
            pltpu.VMEM((CHUNK, DIM), jnp.float32),
            pltpu.SemaphoreType.DMA,
        ],
        compiler_params=pltpu.CompilerParams(use_tc_tiling_on_sc=False),
    )(xt, tbl)
    return tuple(o.reshape(B, T, DIM) for o in outs)


# speedup vs baseline: 2.0657x; 1.3995x over previous
import jax
import jax.numpy as jnp
from jax import lax
from jax.experimental import pallas as pl
from jax.experimental.pallas import tpu as pltpu
from jax.experimental.pallas import tpu_sc as plsc

N_CATS = 26
N_PLAIN = 22
N_TABLES = 23
VOCAB = 100000
DIM = 32
B, T = 1024, 50
GROUP_COLS = (22, 23, 24, 25)

NC, NS = 2, 16
NW = NC * NS
POS = B * T
CHUNK = POS // NW


def _body(xb_hbm, tbl_hbm, *refs):
    outs = refs[:N_TABLES]
    ib0, ib1, ra, rb, isem, gsem, wsem = refs[N_TABLES:]
    ib = (ib0, ib1)
    rows = (ra, rb)
    sid = lax.axis_index("s")
    wid = sid * NC + lax.axis_index("c")
    base = wid * CHUNK

    def start_idx(f):
        return pltpu.async_copy(
            xb_hbm.at[pl.ds(f * POS + base, CHUNK)], ib[f % 2], isem)

    def gslot(f):
        return 1 if f > N_PLAIN else f % 2

    def start_gather(f):
        return pltpu.async_copy(tbl_hbm.at[ib[f % 2]], rows[gslot(f)], gsem)

    def start_wb(f):
        return pltpu.async_copy(
            rows[f % 2], outs[f].at[pl.ds(base, CHUNK)], wsem)

    g, w, di = {}, {}, {}
    start_idx(0).wait()
    g[0] = start_gather(0)
    di[1] = start_idx(1)
    for f in range(N_PLAIN + 1):
        g[f].wait()
        if f + 1 < N_CATS:
            di[f + 1].wait()
            if f >= 1 and f - 1 < N_PLAIN:
                w[f - 1].wait()
            g[f + 1] = start_gather(f + 1)
        if f + 2 < N_CATS:
            di[f + 2] = start_idx(f + 2)
        if f < N_PLAIN:
            w[f] = start_wb(f)
    def add_row(i, _):
        for k in range(DIM // 16):
            sl = pl.ds(k * 16, 16)
            ra[i, sl] = ra[i, sl] + rb[i, sl]
        return 0

    for f in range(N_PLAIN + 1, N_CATS):
        g[f].wait()
        if f + 2 < N_CATS:
            di[f + 2] = start_idx(f + 2)
        lax.fori_loop(0, CHUNK, add_row, 0)
        if f + 1 < N_CATS:
            di[f + 1].wait()
            g[f + 1] = start_gather(f + 1)
    pltpu.sync_copy(ra, outs[N_PLAIN].at[pl.ds(base, CHUNK)])


@jax.jit
def kernel(x, tables):
    xb = jnp.transpose(x, (2, 1, 0)).reshape(N_CATS, POS)
    tid = jnp.array(list(range(N_PLAIN)) + [N_PLAIN] * 4, dtype=jnp.int32)
    xb = (xb + tid[:, None] * VOCAB).reshape(-1)
    tbl = tables.reshape(N_TABLES * VOCAB, DIM)

    mesh = plsc.VectorSubcoreMesh(core_axis_name="c", subcore_axis_name="s")
    out_type = tuple(
        jax.ShapeDtypeStruct((POS, DIM), jnp.float32) for _ in range(N_TABLES)
    )
    outs = pl.kernel(
        _body,
        out_type=out_type,
        mesh=mesh,
        scratch_types=[
            pltpu.VMEM((CHUNK,), jnp.int32),
            pltpu.VMEM((CHUNK,), jnp.int32),
            pltpu.VMEM((CHUNK, DIM), jnp.float32),
            pltpu.VMEM((CHUNK, DIM), jnp.float32),
            pltpu.SemaphoreType.DMA,
            pltpu.SemaphoreType.DMA,
            pltpu.SemaphoreType.DMA,
        ],
        compiler_params=pltpu.CompilerParams(use_tc_tiling_on_sc=False),
    )(xb, tbl)
    return tuple(
        o.reshape(T, B, DIM).transpose(1, 0, 2) for o in outs
    )
